# Initial kernel scaffold; baseline (speedup 1.0000x reference)
import jax
import jax.numpy as jnp
from jax.experimental import pallas as pl


def kernel(x, edge_index, edge_attr, eps, geW, geb, gW1, gb1, gW2, gb2, n1g, n1b, gatW, asrc, adst, gbias, n2g, n2b):
    raise NotImplementedError("write your pallas kernel here")


# speedup vs baseline: 4.2142x; 4.2142x over previous
import functools

import jax
import jax.numpy as jnp
from jax import lax
from jax.experimental import pallas as pl
from jax.experimental.pallas import tpu as pltpu
from jax.experimental.pallas import tpu_sc as plsc

_NC = 2
_NS = 16
_CHA = 64
_CH = 128


def _leaky(z):
    return jnp.where(z >= 0.0, z, 0.2 * z)


def _mesh():
    return plsc.VectorSubcoreMesh(
        core_axis_name="c", subcore_axis_name="s", num_cores=_NC,
        num_subcores=_NS)


def _edge_linear(edge_attr, geW, geb):
    E, D = edge_attr.shape
    BE = 3200
    grid = E // BE

    def body(ea, w, b, o):
        o[...] = (
            jnp.dot(ea[...], w[...], preferred_element_type=jnp.float32) + b[...]
        )

    return pl.pallas_call(
        body,
        grid=(grid,),
        in_specs=[
            pl.BlockSpec((BE, D), lambda i: (i, 0)),
            pl.BlockSpec((D, D), lambda i: (0, 0)),
            pl.BlockSpec((1, D), lambda i: (0, 0)),
        ],
        out_specs=pl.BlockSpec((BE, D), lambda i: (i, 0)),
        out_shape=jax.ShapeDtypeStruct((E, D), jnp.float32),
    )(edge_attr, geW, geb.reshape(1, D))


def _zero_fill(dst_ref, zbuf, off, total):
    done = 0
    zrows = zbuf.shape[0]
    while done < total:
        step = min(zrows, total - done)
        pltpu.sync_copy(zbuf.at[pl.ds(0, step)],
                        dst_ref.at[pl.ds(pl.multiple_of(off + done, 8), step)])
        done += step


def _gine_aggregate(x, e, src, dst):
    N, D = x.shape
    E = e.shape[0]
    nchunks = E // _CHA
    NQ = N // (2 * _NC)
    rpt = (NQ // _NS) // 8 * 8
    rem = NQ - _NS * rpt

    @functools.partial(
        pl.kernel,
        out_type=jax.ShapeDtypeStruct((_NC, 2, NQ, D), jnp.float32),
        mesh=_mesh(),
        scratch_types=[
            pltpu.VMEM((_CHA,), jnp.int32),
            pltpu.VMEM((_CHA,), jnp.int32),
            pltpu.VMEM((_CHA, D), jnp.float32),
            pltpu.VMEM((_CHA, D), jnp.float32),
            pltpu.VMEM_SHARED((NQ + 8, D), jnp.float32),
            pltpu.SemaphoreType.DMA,
        ],
    )
    def k(x_hbm, e_hbm, src_hbm, dst_hbm, out_hbm,
          sidx, didx, ebuf, xbuf, acc, sem):
        c = lax.axis_index("c")
        s = lax.axis_index("s")
        off = pl.multiple_of(s * rpt, 8)
        ntr = (nchunks - s + _NS - 1) // _NS

        for q in range(2):
            lo = c * (2 * NQ) + q * NQ

            def zrow(i, _):
                for j in range(D // 16):
                    ebuf[i, pl.ds(j * 16, 16)] = jnp.zeros((16,), jnp.float32)
                return 0

            lax.fori_loop(0, _CHA, zrow, 0)
            _zero_fill(acc, ebuf, off, rpt)

            @pl.when(s == _NS - 1)
            def _():
                _zero_fill(acc, ebuf, _NS * rpt, (NQ + 8) - _NS * rpt)

            plsc.subcore_barrier()

            def chunk(kk, _):
                base = pl.multiple_of((s + kk * _NS) * _CHA, _CHA)
                pltpu.sync_copy(src_hbm.at[pl.ds(base, _CHA)], sidx)
                pltpu.sync_copy(dst_hbm.at[pl.ds(base, _CHA)], didx)
                gat = pltpu.async_copy(x_hbm.at[sidx], xbuf, sem)
                pltpu.sync_copy(e_hbm.at[pl.ds(base, _CHA)], ebuf)
                for g in range(_CHA // 16):
                    sl = pl.ds(g * 16, 16)
                    lv = didx[sl] - lo
                    oob = (lv < 0) | (lv >= NQ)
                    didx[sl] = jnp.where(oob, NQ, lv)
                gat.wait()

                def row(i, _2):
                    for j in range(D // 16):
                        sl = pl.ds(j * 16, 16)
                        ebuf[i, sl] = jnp.maximum(
                            ebuf[i, sl] + xbuf[i, sl], 0.0)
                    return 0

                lax.fori_loop(0, _CHA, row, 0)
                pltpu.sync_copy(ebuf, acc.at[didx], add=True)
                return 0

            lax.fori_loop(0, ntr, chunk, 0)
            plsc.subcore_barrier()
            pltpu.sync_copy(acc.at[pl.ds(off, rpt)],
                            out_hbm.at[c, q, pl.ds(off, rpt)])

            @pl.when(s == _NS - 1)
            def _():
                pltpu.sync_copy(acc.at[pl.ds(_NS * rpt, rem)],
                                out_hbm.at[c, q, pl.ds(_NS * rpt, rem)])

            plsc.subcore_barrier()

    return k(x, e, src, dst)


def _node_stage(x, aggr, eps, gW1, gb1, gW2, gb2, n1g, n1b, gatW, AS):
    N, D = x.shape
    BN = 2000
    grid = N // BN
    A8 = AS.shape[1]

    def body(eps_r, x_r, ag_r, w1, b1, w2, b2, g1, bb1, gw, as_r,
             h_o, xh_o, asd_o, gmax_o):
        i = pl.program_id(0)
        xv = x_r[...]
        hpre = (1.0 + eps_r[0, 0]) * xv + ag_r[...]
        t = jnp.maximum(
            jnp.dot(hpre, w1[...], preferred_element_type=jnp.float32)
            + b1[...], 0.0)
        h2 = jnp.dot(t, w2[...], preferred_element_type=jnp.float32) + b2[...]
        r = jnp.maximum(h2, 0.0)
        m = jnp.mean(r, axis=-1, keepdims=True)
        v = jnp.mean((r - m) * (r - m), axis=-1, keepdims=True)
        hn = (r - m) / jnp.sqrt(v + 1e-5) * g1[...] + bb1[...]
        h = hn + xv
        h_o[...] = h
        xh = jnp.dot(h, gw[...], preferred_element_type=jnp.float32)
        xh_o[...] = xh
        asd = jnp.dot(xh, as_r[...], preferred_element_type=jnp.float32)
        asd_o[...] = asd
        blkmax = jnp.max(asd, axis=0, keepdims=True)

        @pl.when(i == 0)
        def _():
            gmax_o[...] = jnp.full((1, A8), -1e30, jnp.float32)

        gmax_o[...] = jnp.maximum(gmax_o[...], blkmax)

    return pl.pallas_call(
        body,
        grid=(grid,),
        in_specs=[
            pl.BlockSpec(memory_space=pltpu.SMEM),
            pl.BlockSpec((BN, D), lambda i: (i, 0)),
            pl.BlockSpec((BN, D), lambda i: (i, 0)),
            pl.BlockSpec((D, D), lambda i: (0, 0)),
            pl.BlockSpec((1, D), lambda i: (0, 0)),
            pl.BlockSpec((D, D), lambda i: (0, 0)),
            pl.BlockSpec((1, D), lambda i: (0, 0)),
            pl.BlockSpec((1, D), lambda i: (0, 0)),
            pl.BlockSpec((1, D), lambda i: (0, 0)),
            pl.BlockSpec((D, D), lambda i: (0, 0)),
            pl.BlockSpec((D, A8), lambda i: (0, 0)),
        ],
        out_specs=[
            pl.BlockSpec((BN, D), lambda i: (i, 0)),
            pl.BlockSpec((BN, D), lambda i: (i, 0)),
            pl.BlockSpec((BN, A8), lambda i: (i, 0)),
            pl.BlockSpec((1, A8), lambda i: (0, 0)),
        ],
        out_shape=[
            jax.ShapeDtypeStruct((N, D), jnp.float32),
            jax.ShapeDtypeStruct((N, D), jnp.float32),
            jax.ShapeDtypeStruct((N, A8), jnp.float32),
            jax.ShapeDtypeStruct((1, A8), jnp.float32),
        ],
    )(eps.reshape(1, 1), x, aggr, gW1, gb1.reshape(1, D), gW2,
      gb2.reshape(1, D), n1g.reshape(1, D), n1b.reshape(1, D), gatW, AS)


def _gat_edges(xh, asd128, gmax16, src, dst, H):
    N, D = xh.shape
    E = src.shape[0]
    nchunks = E // _CH
    NQ = N // (2 * _NC)
    rpt = (NQ // _NS) // 8 * 8
    rem = NQ - _NS * rpt
    ND4 = NQ * 4 + 16

    @functools.partial(
        pl.kernel,
        out_type=(
            jax.ShapeDtypeStruct((_NC, 2, NQ, D), jnp.float32),
            jax.ShapeDtypeStruct((_NC, 2, _NS, ND4), jnp.float32),
        ),
        mesh=_mesh(),
        scratch_types=[
            pltpu.VMEM((_CH,), jnp.int32),
            pltpu.VMEM((_CH,), jnp.int32),
            pltpu.VMEM((_CH,), jnp.int32),
            pltpu.VMEM((_CH, D), jnp.float32),
            pltpu.VMEM((_CH, 4), jnp.float32),
            pltpu.VMEM((_CH, D), jnp.float32),
            pltpu.VMEM((_CH, D), jnp.float32),
            pltpu.VMEM((16,), jnp.float32),
            pltpu.VMEM((ND4,), jnp.float32),
            pltpu.VMEM_SHARED((NQ + 8, D), jnp.float32),
            pltpu.SemaphoreType.DMA,
            pltpu.SemaphoreType.DMA,
        ],
        compiler_params=pltpu.CompilerParams(needs_layout_passes=False),
    )
    def k(xh_hbm, asd_hbm, gm_hbm, src_hbm, dst_hbm,
          out_hbm, den_hbm,
          sidx, didx, didx2, xbuf, stg, arow_s, arow_d, gm_v, lden, acc,
          sem, sem2):
        c = lax.axis_index("c")
        s = lax.axis_index("s")
        off = pl.multiple_of(s * rpt, 8)
        pltpu.sync_copy(gm_hbm, gm_v)

        iota16 = lax.iota(jnp.int32, 16)
        zeros16 = jnp.zeros((16,), jnp.float32)
        ntr = (nchunks - s + _NS - 1) // _NS
        gmv = gm_v[...]

        def zxbuf(g, _):
            rows = iota16 + g * 16
            for cj in range(D):
                plsc.store_scatter(
                    xbuf, [rows, jnp.full((16,), cj, jnp.int32)], zeros16)
            return 0

        for q in range(2):
            lo = c * (2 * NQ) + q * NQ

            lax.fori_loop(0, _CH // 16, zxbuf, 0)
            _zero_fill(acc, xbuf, off, rpt)

            @pl.when(s == _NS - 1)
            def _():
                _zero_fill(acc, xbuf, _NS * rpt, (NQ + 8) - _NS * rpt)

            def zden(g, _):
                lden[pl.ds(pl.multiple_of(g * 16, 16), 16)] = zeros16
                return 0

            lax.fori_loop(0, ND4 // 16, zden, 0)
            plsc.subcore_barrier()

            def chunk(kk, _):
                base = pl.multiple_of((s + kk * _NS) * _CH, _CH)
                pltpu.sync_copy(src_hbm.at[pl.ds(base, _CH)], sidx)
                pltpu.sync_copy(dst_hbm.at[pl.ds(base, _CH)], didx)
                ga = pltpu.async_copy(asd_hbm.at[sidx], arow_s, sem2)
                gb = pltpu.async_copy(asd_hbm.at[didx], arow_d, sem2)
                gat = pltpu.async_copy(xh_hbm.at[sidx], xbuf, sem)
                for g in range(_CH // 16):
                    sl = pl.ds(g * 16, 16)
                    lv = didx[sl] - lo
                    oob = (lv < 0) | (lv >= NQ)
                    didx2[sl] = jnp.where(oob, NQ, lv)
                ga.wait()
                gb.wait()

                def exgrp(g, _2):
                    goff = pl.multiple_of(g * 16, 16)
                    rows = iota16 + g * 16
                    lv = didx2[pl.ds(goff, 16)]
                    for h in range(H):
                        va = plsc.load_gather(
                            arow_s, [rows, jnp.full((16,), h, jnp.int32)])
                        vb = plsc.load_gather(
                            arow_d, [rows, jnp.full((16,), H + h, jnp.int32)])
                        zl = _leaky(va + vb)
                        kl = _leaky(gmv[h] + vb)
                        ex = jnp.exp(zl - kl)
                        plsc.store_scatter(
                            stg, [rows, jnp.full((16,), h, jnp.int32)], ex)
                        plsc.addupdate_scatter(lden, [lv * 4 + h], ex)
                    return 0

                lax.fori_loop(0, _CH // 16, exgrp, 0)
                gat.wait()

                def scale(g, _2):
                    rows = iota16 + g * 16
                    for h in range(H):
                        att = plsc.load_gather(
                            stg, [rows, jnp.full((16,), h, jnp.int32)])
                        for cj in range(D // H):
                            cs = jnp.full((16,), h * (D // H) + cj, jnp.int32)
                            v = plsc.load_gather(xbuf, [rows, cs])
                            plsc.store_scatter(xbuf, [rows, cs], v * att)
                    return 0

                lax.fori_loop(0, _CH // 16, scale, 0)
                pltpu.sync_copy(xbuf, acc.at[didx2], add=True)
                return 0

            lax.fori_loop(0, ntr, chunk, 0)
            plsc.subcore_barrier()
            pltpu.sync_copy(acc.at[pl.ds(off, rpt)],
                            out_hbm.at[c, q, pl.ds(off, rpt)])

            @pl.when(s == _NS - 1)
            def _():
                pltpu.sync_copy(acc.at[pl.ds(_NS * rpt, rem)],
                                out_hbm.at[c, q, pl.ds(_NS * rpt, rem)])

            pltpu.sync_copy(lden, den_hbm.at[c, q, s])
            plsc.subcore_barrier()

    return k(xh, asd128, gmax16, src, dst)


def _final(acc, dparts, asd, gmax, xh, h, gbias, n2g, n2b, R4, R16, H):
    N, D = xh.shape
    BN = 2000
    grid = N // BN
    A8 = asd.shape[1]
    P = dparts.shape[1]

    def body(a0, d0, asd_r, gm_r, xh_r, h_r, gb, g2, b2, r4, r16, o):
        asd_v = asd_r[...]
        a_s = lax.slice(asd_v, (0, 0), (BN, H))
        a_d = lax.slice(asd_v, (0, H), (BN, 2 * H))
        gm = lax.slice(gm_r[...], (0, 0), (1, H))
        ex = jnp.exp(_leaky(a_s + a_d) - _leaky(gm + a_d))
        den4 = jnp.dot(d0[...], r16[...],
                       preferred_element_type=jnp.float32) + ex
        r4v = r4[...]
        ex128 = jnp.dot(ex, r4v, preferred_element_type=jnp.float32)
        den128 = jnp.dot(den4, r4v, preferred_element_type=jnp.float32)
        out = a0[...] + ex128 * xh_r[...]
        g1 = out / (den128 + 1e-16) + gb[...]
        r = jnp.maximum(g1, 0.0)
        m = jnp.mean(r, axis=-1, keepdims=True)
        v = jnp.mean((r - m) * (r - m), axis=-1, keepdims=True)
        o[...] = (r - m) / jnp.sqrt(v + 1e-5) * g2[...] + b2[...] + h_r[...]

    return pl.pallas_call(
        body,
        grid=(grid,),
        in_specs=[
            pl.BlockSpec((BN, D), lambda i: (i, 0)),
            pl.BlockSpec((BN, P), lambda i: (i, 0)),
            pl.BlockSpec((BN, A8), lambda i: (i, 0)),
            pl.BlockSpec((1, A8), lambda i: (0, 0)),
            pl.BlockSpec((BN, D), lambda i: (i, 0)),
            pl.BlockSpec((BN, D), lambda i: (i, 0)),
            pl.BlockSpec((1, D), lambda i: (0, 0)),
            pl.BlockSpec((1, D), lambda i: (0, 0)),
            pl.BlockSpec((1, D), lambda i: (0, 0)),
            pl.BlockSpec((H, D), lambda i: (0, 0)),
            pl.BlockSpec((P, H), lambda i: (0, 0)),
        ],
        out_specs=pl.BlockSpec((BN, D), lambda i: (i, 0)),
        out_shape=jax.ShapeDtypeStruct((N, D), jnp.float32),
    )(acc, dparts, asd, gmax, xh, h, gbias.reshape(1, D),
      n2g.reshape(1, D), n2b.reshape(1, D), R4, R16)


def kernel(x, edge_index, edge_attr, eps, geW, geb, gW1, gb1, gW2, gb2,
           n1g, n1b, gatW, asrc, adst, gbias, n2g, n2b):
    N, D = x.shape
    H, C = asrc.shape
    NH = N // _NC
    src = edge_index[0]
    dst = edge_index[1]

    rows = jnp.arange(D)
    hh = rows // C
    cc = rows % C
    AS = jnp.zeros((D, D), jnp.float32)
    AS = AS.at[rows, hh].set(asrc[hh, cc])
    AS = AS.at[rows, H + hh].set(adst[hh, cc])
    R4 = (jnp.arange(H)[:, None] == hh[None, :]).astype(jnp.float32)
    R16 = jnp.tile(jnp.eye(H, dtype=jnp.float32), (_NS, 1))

    e = _edge_linear(edge_attr, geW, geb)
    aggr2 = _gine_aggregate(x, e, src, dst)
    aggr = aggr2.reshape(N, D)
    h, xh, asd, gmax = _node_stage(x, aggr, eps, gW1, gb1, gW2,
                                   gb2, n1g, n1b, gatW, AS)
    gmax16 = gmax[0, :16]
    outp, denp = _gat_edges(xh, asd, gmax16, src, dst, H)
    outacc = outp.reshape(N, D)
    NQ = N // (2 * _NC)
    dparts = jnp.transpose(
        denp[:, :, :, :NQ * 4].reshape(_NC, 2, _NS, NQ, H), (0, 1, 3, 2, 4)
    ).reshape(N, _NS * H)
    return _final(outacc, dparts, asd, gmax, xh, h, gbias, n2g, n2b,
                  R4, R16, H)


# speedup vs baseline: 6.8143x; 1.6170x over previous
import functools

import jax
import jax.numpy as jnp
from jax import lax
from jax.experimental import pallas as pl
from jax.experimental.pallas import tpu as pltpu
from jax.experimental.pallas import tpu_sc as plsc

_NC = 2
_NS = 16
_CHA = 64
_CH = 64


def _leaky(z):
    return jnp.where(z >= 0.0, z, 0.2 * z)


def _mesh():
    return plsc.VectorSubcoreMesh(
        core_axis_name="c", subcore_axis_name="s", num_cores=_NC,
        num_subcores=_NS)


def _edge_linear(edge_attr, geW, geb):
    E, D = edge_attr.shape
    BE = 3200
    grid = E // BE

    def body(ea, w, b, o):
        o[...] = (
            jnp.dot(ea[...], w[...], preferred_element_type=jnp.float32) + b[...]
        )

    return pl.pallas_call(
        body,
        grid=(grid,),
        in_specs=[
            pl.BlockSpec((BE, D), lambda i: (i, 0)),
            pl.BlockSpec((D, D), lambda i: (0, 0)),
            pl.BlockSpec((1, D), lambda i: (0, 0)),
        ],
        out_specs=pl.BlockSpec((BE, D), lambda i: (i, 0)),
        out_shape=jax.ShapeDtypeStruct((E, D), jnp.float32),
    )(edge_attr, geW, geb.reshape(1, D))


def _zero_fill(dst_ref, zbuf, off, total):
    done = 0
    zrows = zbuf.shape[0]
    while done < total:
        step = min(zrows, total - done)
        pltpu.sync_copy(zbuf.at[pl.ds(0, step)],
                        dst_ref.at[pl.ds(pl.multiple_of(off + done, 8), step)])
        done += step


def _gine_aggregate(x, e, src, dst):
    N, D = x.shape
    E = e.shape[0]
    nchunks = E // _CHA
    NQ = N // (2 * _NC)
    rpt = (NQ // _NS) // 8 * 8
    rem = NQ - _NS * rpt

    @functools.partial(
        pl.kernel,
        out_type=jax.ShapeDtypeStruct((_NC, 2, NQ, D), jnp.float32),
        mesh=_mesh(),
        scratch_types=[
            pltpu.VMEM((_CHA,), jnp.int32),
            pltpu.VMEM((_CHA,), jnp.int32),
            pltpu.VMEM((_CHA, D), jnp.float32),
            pltpu.VMEM((_CHA, D), jnp.float32),
            pltpu.VMEM_SHARED((NQ + 8, D), jnp.float32),
            pltpu.SemaphoreType.DMA,
        ],
    )
    def k(x_hbm, e_hbm, src_hbm, dst_hbm, out_hbm,
          sidx, didx, ebuf, xbuf, acc, sem):
        c = lax.axis_index("c")
        s = lax.axis_index("s")
        off = pl.multiple_of(s * rpt, 8)
        ntr = (nchunks - s + _NS - 1) // _NS

        for q in range(2):
            lo = c * (2 * NQ) + q * NQ

            def zrow(i, _):
                for j in range(D // 16):
                    ebuf[i, pl.ds(j * 16, 16)] = jnp.zeros((16,), jnp.float32)
                return 0

            lax.fori_loop(0, _CHA, zrow, 0)
            _zero_fill(acc, ebuf, off, rpt)

            @pl.when(s == _NS - 1)
            def _():
                _zero_fill(acc, ebuf, _NS * rpt, (NQ + 8) - _NS * rpt)

            plsc.subcore_barrier()

            def chunk(kk, _):
                base = pl.multiple_of((s + kk * _NS) * _CHA, _CHA)
                pltpu.sync_copy(src_hbm.at[pl.ds(base, _CHA)], sidx)
                pltpu.sync_copy(dst_hbm.at[pl.ds(base, _CHA)], didx)
                gat = pltpu.async_copy(x_hbm.at[sidx], xbuf, sem)
                pltpu.sync_copy(e_hbm.at[pl.ds(base, _CHA)], ebuf)
                for g in range(_CHA // 16):
                    sl = pl.ds(g * 16, 16)
                    lv = didx[sl] - lo
                    oob = (lv < 0) | (lv >= NQ)
                    didx[sl] = jnp.where(oob, NQ, lv)
                gat.wait()

                def row(i, _2):
                    for j in range(D // 16):
                        sl = pl.ds(j * 16, 16)
                        ebuf[i, sl] = jnp.maximum(
                            ebuf[i, sl] + xbuf[i, sl], 0.0)
                    return 0

                lax.fori_loop(0, _CHA, row, 0)
                pltpu.sync_copy(ebuf, acc.at[didx], add=True)
                return 0

            lax.fori_loop(0, ntr, chunk, 0)
            plsc.subcore_barrier()
            pltpu.sync_copy(acc.at[pl.ds(off, rpt)],
                            out_hbm.at[c, q, pl.ds(off, rpt)])

            @pl.when(s == _NS - 1)
            def _():
                pltpu.sync_copy(acc.at[pl.ds(_NS * rpt, rem)],
                                out_hbm.at[c, q, pl.ds(_NS * rpt, rem)])

            plsc.subcore_barrier()

    return k(x, e, src, dst)


def _node_stage(x, aggr, eps, gW1, gb1, gW2, gb2, n1g, n1b, gatW, AS):
    N, D = x.shape
    BN = 2000
    grid = N // BN
    A8 = AS.shape[1]

    def body(eps_r, x_r, ag_r, w1, b1, w2, b2, g1, bb1, gw, as_r,
             h_o, xh_o, asd_o, gmax_o):
        i = pl.program_id(0)
        xv = x_r[...]
        hpre = (1.0 + eps_r[0, 0]) * xv + ag_r[...]
        t = jnp.maximum(
            jnp.dot(hpre, w1[...], preferred_element_type=jnp.float32)
            + b1[...], 0.0)
        h2 = jnp.dot(t, w2[...], preferred_element_type=jnp.float32) + b2[...]
        r = jnp.maximum(h2, 0.0)
        m = jnp.mean(r, axis=-1, keepdims=True)
        v = jnp.mean((r - m) * (r - m), axis=-1, keepdims=True)
        hn = (r - m) / jnp.sqrt(v + 1e-5) * g1[...] + bb1[...]
        h = hn + xv
        h_o[...] = h
        xh = jnp.dot(h, gw[...], preferred_element_type=jnp.float32)
        xh_o[...] = xh
        asd = jnp.dot(xh, as_r[...], preferred_element_type=jnp.float32)
        asd_o[...] = asd
        blkmax = jnp.max(asd, axis=0, keepdims=True)

        @pl.when(i == 0)
        def _():
            gmax_o[...] = jnp.full((1, A8), -1e30, jnp.float32)

        gmax_o[...] = jnp.maximum(gmax_o[...], blkmax)

    return pl.pallas_call(
        body,
        grid=(grid,),
        in_specs=[
            pl.BlockSpec(memory_space=pltpu.SMEM),
            pl.BlockSpec((BN, D), lambda i: (i, 0)),
            pl.BlockSpec((BN, D), lambda i: (i, 0)),
            pl.BlockSpec((D, D), lambda i: (0, 0)),
            pl.BlockSpec((1, D), lambda i: (0, 0)),
            pl.BlockSpec((D, D), lambda i: (0, 0)),
            pl.BlockSpec((1, D), lambda i: (0, 0)),
            pl.BlockSpec((1, D), lambda i: (0, 0)),
            pl.BlockSpec((1, D), lambda i: (0, 0)),
            pl.BlockSpec((D, D), lambda i: (0, 0)),
            pl.BlockSpec((D, A8), lambda i: (0, 0)),
        ],
        out_specs=[
            pl.BlockSpec((BN, D), lambda i: (i, 0)),
            pl.BlockSpec((BN, D), lambda i: (i, 0)),
            pl.BlockSpec((BN, A8), lambda i: (i, 0)),
            pl.BlockSpec((1, A8), lambda i: (0, 0)),
        ],
        out_shape=[
            jax.ShapeDtypeStruct((N, D), jnp.float32),
            jax.ShapeDtypeStruct((N, D), jnp.float32),
            jax.ShapeDtypeStruct((N, A8), jnp.float32),
            jax.ShapeDtypeStruct((1, A8), jnp.float32),
        ],
    )(eps.reshape(1, 1), x, aggr, gW1, gb1.reshape(1, D), gW2,
      gb2.reshape(1, D), n1g.reshape(1, D), n1b.reshape(1, D), gatW, AS)


def _gat_edges(xh, asd128, gmax16, src, dst, H):
    N, D = xh.shape
    E = src.shape[0]
    nchunks = E // _CH
    NQ = N // _NC
    rpt = (NQ // _NS) // 8 * 8
    rem = NQ - _NS * rpt
    ND4 = NQ * 4 + 16

    @functools.partial(
        pl.kernel,
        out_type=(
            jax.ShapeDtypeStruct((_NC, 1, NQ, D), jnp.float32),
            jax.ShapeDtypeStruct((_NC, 1, _NS, ND4), jnp.float32),
        ),
        mesh=_mesh(),
        scratch_types=[
            pltpu.VMEM((_CH,), jnp.int32),
            pltpu.VMEM((_CH,), jnp.int32),
            pltpu.VMEM((_CH,), jnp.int32),
            pltpu.VMEM((_CH, D), jnp.float32),
            pltpu.VMEM((_CH, 4), jnp.float32),
            pltpu.VMEM((_CH, D), jnp.float32),
            pltpu.VMEM((_CH, D), jnp.float32),
            pltpu.VMEM((16,), jnp.float32),
            pltpu.VMEM((ND4,), jnp.float32),
            pltpu.VMEM_SHARED((NQ + 8, D), jnp.float32),
            pltpu.SemaphoreType.DMA,
            pltpu.SemaphoreType.DMA,
        ],
        compiler_params=pltpu.CompilerParams(needs_layout_passes=False),
    )
    def k(xh_hbm, asd_hbm, gm_hbm, src_hbm, dst_hbm,
          out_hbm, den_hbm,
          sidx, didx, didx2, xbuf, stg, arow_s, arow_d, gm_v, lden, acc,
          sem, sem2):
        c = lax.axis_index("c")
        s = lax.axis_index("s")
        off = pl.multiple_of(s * rpt, 8)
        pltpu.sync_copy(gm_hbm, gm_v)

        iota16 = lax.iota(jnp.int32, 16)
        zeros16 = jnp.zeros((16,), jnp.float32)
        ntr = (nchunks - s + _NS - 1) // _NS
        gmv = gm_v[...]

        def zxbuf(g, _):
            rows = iota16 + g * 16
            for cj in range(D):
                plsc.store_scatter(
                    xbuf, [rows, jnp.full((16,), cj, jnp.int32)], zeros16)
            return 0

        for q in range(1):
            lo = c * NQ

            lax.fori_loop(0, _CH // 16, zxbuf, 0)
            _zero_fill(acc, xbuf, off, rpt)

            @pl.when(s == _NS - 1)
            def _():
                _zero_fill(acc, xbuf, _NS * rpt, (NQ + 8) - _NS * rpt)

            def zden(g, _):
                lden[pl.ds(pl.multiple_of(g * 16, 16), 16)] = zeros16
                return 0

            lax.fori_loop(0, ND4 // 16, zden, 0)
            plsc.subcore_barrier()

            def chunk(kk, _):
                base = pl.multiple_of((s + kk * _NS) * _CH, _CH)
                pltpu.sync_copy(src_hbm.at[pl.ds(base, _CH)], sidx)
                pltpu.sync_copy(dst_hbm.at[pl.ds(base, _CH)], didx)
                ga = pltpu.async_copy(asd_hbm.at[sidx], arow_s, sem2)
                gb = pltpu.async_copy(asd_hbm.at[didx], arow_d, sem2)
                gat = pltpu.async_copy(xh_hbm.at[sidx], xbuf, sem)
                for g in range(_CH // 16):
                    sl = pl.ds(g * 16, 16)
                    lv = didx[sl] - lo
                    oob = (lv < 0) | (lv >= NQ)
                    didx2[sl] = jnp.where(oob, NQ, lv)
                ga.wait()
                gb.wait()

                def exgrp(g, _2):
                    goff = pl.multiple_of(g * 16, 16)
                    rows = iota16 + g * 16
                    lv = didx2[pl.ds(goff, 16)]
                    for h in range(H):
                        va = plsc.load_gather(
                            arow_s, [rows, jnp.full((16,), h, jnp.int32)])
                        vb = plsc.load_gather(
                            arow_d, [rows, jnp.full((16,), H + h, jnp.int32)])
                        zl = _leaky(va + vb)
                        kl = _leaky(gmv[h] + vb)
                        ex = jnp.exp(zl - kl)
                        plsc.store_scatter(
                            stg, [rows, jnp.full((16,), h, jnp.int32)], ex)
                        plsc.addupdate_scatter(lden, [lv * 4 + h], ex)
                    return 0

                lax.fori_loop(0, _CH // 16, exgrp, 0)
                gat.wait()

                def scale(g, _2):
                    rows = iota16 + g * 16
                    for h in range(H):
                        att = plsc.load_gather(
                            stg, [rows, jnp.full((16,), h, jnp.int32)])
                        for cj in range(D // H):
                            cs = jnp.full((16,), h * (D // H) + cj, jnp.int32)
                            v = plsc.load_gather(xbuf, [rows, cs])
                            plsc.store_scatter(xbuf, [rows, cs], v * att)
                    return 0

                lax.fori_loop(0, _CH // 16, scale, 0)
                pltpu.sync_copy(xbuf, acc.at[didx2], add=True)
                return 0

            lax.fori_loop(0, ntr, chunk, 0)
            plsc.subcore_barrier()
            pltpu.sync_copy(acc.at[pl.ds(off, rpt)],
                            out_hbm.at[c, q, pl.ds(off, rpt)])

            @pl.when(s == _NS - 1)
            def _():
                pltpu.sync_copy(acc.at[pl.ds(_NS * rpt, rem)],
                                out_hbm.at[c, q, pl.ds(_NS * rpt, rem)])

            pltpu.sync_copy(lden, den_hbm.at[c, q, s])
            plsc.subcore_barrier()

    return k(xh, asd128, gmax16, src, dst)


def _final(acc, dparts, asd, gmax, xh, h, gbias, n2g, n2b, R4, R16, H):
    N, D = xh.shape
    BN = 2000
    grid = N // BN
    A8 = asd.shape[1]
    P = dparts.shape[1]

    def body(a0, d0, asd_r, gm_r, xh_r, h_r, gb, g2, b2, r4, r16, o):
        asd_v = asd_r[...]
        a_s = lax.slice(asd_v, (0, 0), (BN, H))
        a_d = lax.slice(asd_v, (0, H), (BN, 2 * H))
        gm = lax.slice(gm_r[...], (0, 0), (1, H))
        ex = jnp.exp(_leaky(a_s + a_d) - _leaky(gm + a_d))
        den4 = jnp.dot(d0[...], r16[...],
                       preferred_element_type=jnp.float32) + ex
        r4v = r4[...]
        ex128 = jnp.dot(ex, r4v, preferred_element_type=jnp.float32)
        den128 = jnp.dot(den4, r4v, preferred_element_type=jnp.float32)
        out = a0[...] + ex128 * xh_r[...]
        g1 = out / (den128 + 1e-16) + gb[...]
        r = jnp.maximum(g1, 0.0)
        m = jnp.mean(r, axis=-1, keepdims=True)
        v = jnp.mean((r - m) * (r - m), axis=-1, keepdims=True)
        o[...] = (r - m) / jnp.sqrt(v + 1e-5) * g2[...] + b2[...] + h_r[...]

    return pl.pallas_call(
        body,
        grid=(grid,),
        in_specs=[
            pl.BlockSpec((BN, D), lambda i: (i, 0)),
            pl.BlockSpec((BN, P), lambda i: (i, 0)),
            pl.BlockSpec((BN, A8), lambda i: (i, 0)),
            pl.BlockSpec((1, A8), lambda i: (0, 0)),
            pl.BlockSpec((BN, D), lambda i: (i, 0)),
            pl.BlockSpec((BN, D), lambda i: (i, 0)),
            pl.BlockSpec((1, D), lambda i: (0, 0)),
            pl.BlockSpec((1, D), lambda i: (0, 0)),
            pl.BlockSpec((1, D), lambda i: (0, 0)),
            pl.BlockSpec((H, D), lambda i: (0, 0)),
            pl.BlockSpec((P, H), lambda i: (0, 0)),
        ],
        out_specs=pl.BlockSpec((BN, D), lambda i: (i, 0)),
        out_shape=jax.ShapeDtypeStruct((N, D), jnp.float32),
    )(acc, dparts, asd, gmax, xh, h, gbias.reshape(1, D),
      n2g.reshape(1, D), n2b.reshape(1, D), R4, R16)


def kernel(x, edge_index, edge_attr, eps, geW, geb, gW1, gb1, gW2, gb2,
           n1g, n1b, gatW, asrc, adst, gbias, n2g, n2b):
    N, D = x.shape
    H, C = asrc.shape
    NH = N // _NC
    src = edge_index[0]
    dst = edge_index[1]

    rows = jnp.arange(D)
    hh = rows // C
    cc = rows % C
    AS = jnp.zeros((D, D), jnp.float32)
    AS = AS.at[rows, hh].set(asrc[hh, cc])
    AS = AS.at[rows, H + hh].set(adst[hh, cc])
    R4 = (jnp.arange(H)[:, None] == hh[None, :]).astype(jnp.float32)
    R16 = jnp.tile(jnp.eye(H, dtype=jnp.float32), (_NS, 1))

    e = _edge_linear(edge_attr, geW, geb)
    aggr2 = _gine_aggregate(x, e, src, dst)
    aggr = aggr2.reshape(N, D)
    h, xh, asd, gmax = _node_stage(x, aggr, eps, gW1, gb1, gW2,
                                   gb2, n1g, n1b, gatW, AS)
    gmax16 = gmax[0, :16]
    outp, denp = _gat_edges(xh, asd, gmax16, src, dst, H)
    outacc = outp.reshape(N, D)
    nc_, qp_, ns_, nd4_ = denp.shape
    np_ = (nd4_ - 16) // 4
    dparts = jnp.transpose(
        denp[:, :, :, :np_ * 4].reshape(nc_, qp_, ns_, np_, H),
        (0, 1, 3, 2, 4)
    ).reshape(N, ns_ * H)
    return _final(outacc, dparts, asd, gmax, xh, h, gbias, n2g, n2b,
                  R4, R16, H)


# speedup vs baseline: 6.8179x; 1.0005x over previous
import functools

import jax
import jax.numpy as jnp
from jax import lax
from jax.experimental import pallas as pl
from jax.experimental.pallas import tpu as pltpu
from jax.experimental.pallas import tpu_sc as plsc

_NC = 2
_NS = 16
_CHA = 64
_CH = 64


def _leaky(z):
    return jnp.where(z >= 0.0, z, 0.2 * z)


def _mesh():
    return plsc.VectorSubcoreMesh(
        core_axis_name="c", subcore_axis_name="s", num_cores=_NC,
        num_subcores=_NS)


def _edge_linear(edge_attr, geW, geb):
    E, D = edge_attr.shape
    BE = 3200
    grid = E // BE

    def body(ea, w, b, o):
        o[...] = (
            jnp.dot(ea[...], w[...], preferred_element_type=jnp.float32) + b[...]
        )

    return pl.pallas_call(
        body,
        grid=(grid,),
        in_specs=[
            pl.BlockSpec((BE, D), lambda i: (i, 0)),
            pl.BlockSpec((D, D), lambda i: (0, 0)),
            pl.BlockSpec((1, D), lambda i: (0, 0)),
        ],
        out_specs=pl.BlockSpec((BE, D), lambda i: (i, 0)),
        out_shape=jax.ShapeDtypeStruct((E, D), jnp.float32),
    )(edge_attr, geW, geb.reshape(1, D))


def _zero_fill(dst_ref, zbuf, off, total):
    done = 0
    zrows = zbuf.shape[0]
    while done < total:
        step = min(zrows, total - done)
        pltpu.sync_copy(zbuf.at[pl.ds(0, step)],
                        dst_ref.at[pl.ds(pl.multiple_of(off + done, 8), step)])
        done += step


def _gine_aggregate(x, e, src, dst):
    N, D = x.shape
    E = e.shape[0]
    nchunks = E // _CHA
    NQ = N // (2 * _NC)
    rpt = (NQ // _NS) // 8 * 8
    rem = NQ - _NS * rpt

    @functools.partial(
        pl.kernel,
        out_type=jax.ShapeDtypeStruct((_NC, 2, NQ, D), jnp.float32),
        mesh=_mesh(),
        scratch_types=[
            pltpu.VMEM((_CHA,), jnp.int32),
            pltpu.VMEM((_CHA,), jnp.int32),
            pltpu.VMEM((_CHA, D), jnp.float32),
            pltpu.VMEM((_CHA, D), jnp.float32),
            pltpu.VMEM_SHARED((NQ + 8, D), jnp.float32),
            pltpu.SemaphoreType.DMA,
            pltpu.SemaphoreType.DMA,
        ],
    )
    def k(x_hbm, e_hbm, src_hbm, dst_hbm, out_hbm,
          sidx, didx, ebuf, xbuf, acc, sem, sem3):
        c = lax.axis_index("c")
        s = lax.axis_index("s")
        off = pl.multiple_of(s * rpt, 8)
        ntr = (nchunks - s + _NS - 1) // _NS

        for q in range(2):
            lo = c * (2 * NQ) + q * NQ

            def zrow(i, _):
                for j in range(D // 16):
                    ebuf[i, pl.ds(j * 16, 16)] = jnp.zeros((16,), jnp.float32)
                return 0

            lax.fori_loop(0, _CHA, zrow, 0)
            _zero_fill(acc, ebuf, off, rpt)

            @pl.when(s == _NS - 1)
            def _():
                _zero_fill(acc, ebuf, _NS * rpt, (NQ + 8) - _NS * rpt)

            plsc.subcore_barrier()

            def chunk(kk, _):
                @pl.when(kk > 0)
                def _():
                    pltpu.make_async_copy(
                        e_hbm.at[pl.ds(0, _CHA)], ebuf, sem3).wait()

                base = pl.multiple_of((s + kk * _NS) * _CHA, _CHA)
                pltpu.sync_copy(src_hbm.at[pl.ds(base, _CHA)], sidx)
                pltpu.sync_copy(dst_hbm.at[pl.ds(base, _CHA)], didx)
                gat = pltpu.async_copy(x_hbm.at[sidx], xbuf, sem)
                pltpu.sync_copy(e_hbm.at[pl.ds(base, _CHA)], ebuf)
                for g in range(_CHA // 16):
                    sl = pl.ds(g * 16, 16)
                    lv = didx[sl] - lo
                    oob = (lv < 0) | (lv >= NQ)
                    didx[sl] = jnp.where(oob, NQ, lv)
                gat.wait()

                def row(i, _2):
                    for j in range(D // 16):
                        sl = pl.ds(j * 16, 16)
                        ebuf[i, sl] = jnp.maximum(
                            ebuf[i, sl] + xbuf[i, sl], 0.0)
                    return 0

                lax.fori_loop(0, _CHA, row, 0)
                pltpu.async_copy(ebuf, acc.at[didx], sem3, add=True)
                return 0

            lax.fori_loop(0, ntr, chunk, 0)
            pltpu.make_async_copy(e_hbm.at[pl.ds(0, _CHA)], ebuf, sem3).wait()
            plsc.subcore_barrier()
            pltpu.sync_copy(acc.at[pl.ds(off, rpt)],
                            out_hbm.at[c, q, pl.ds(off, rpt)])

            @pl.when(s == _NS - 1)
            def _():
                pltpu.sync_copy(acc.at[pl.ds(_NS * rpt, rem)],
                                out_hbm.at[c, q, pl.ds(_NS * rpt, rem)])

            plsc.subcore_barrier()

    return k(x, e, src, dst)


def _node_stage(x, aggr, eps, gW1, gb1, gW2, gb2, n1g, n1b, gatW, AS):
    N, D = x.shape
    BN = 2000
    grid = N // BN
    A8 = AS.shape[1]

    def body(eps_r, x_r, ag_r, w1, b1, w2, b2, g1, bb1, gw, as_r,
             h_o, xh_o, asd_o, gmax_o):
        i = pl.program_id(0)
        xv = x_r[...]
        hpre = (1.0 + eps_r[0, 0]) * xv + ag_r[...]
        t = jnp.maximum(
            jnp.dot(hpre, w1[...], preferred_element_type=jnp.float32)
            + b1[...], 0.0)
        h2 = jnp.dot(t, w2[...], preferred_element_type=jnp.float32) + b2[...]
        r = jnp.maximum(h2, 0.0)
        m = jnp.mean(r, axis=-1, keepdims=True)
        v = jnp.mean((r - m) * (r - m), axis=-1, keepdims=True)
        hn = (r - m) / jnp.sqrt(v + 1e-5) * g1[...] + bb1[...]
        h = hn + xv
        h_o[...] = h
        xh = jnp.dot(h, gw[...], preferred_element_type=jnp.float32)
        xh_o[...] = xh
        asd = jnp.dot(xh, as_r[...], preferred_element_type=jnp.float32)
        asd_o[...] = asd
        blkmax = jnp.max(asd, axis=0, keepdims=True)

        @pl.when(i == 0)
        def _():
            gmax_o[...] = jnp.full((1, A8), -1e30, jnp.float32)

        gmax_o[...] = jnp.maximum(gmax_o[...], blkmax)

    return pl.pallas_call(
        body,
        grid=(grid,),
        in_specs=[
            pl.BlockSpec(memory_space=pltpu.SMEM),
            pl.BlockSpec((BN, D), lambda i: (i, 0)),
            pl.BlockSpec((BN, D), lambda i: (i, 0)),
            pl.BlockSpec((D, D), lambda i: (0, 0)),
            pl.BlockSpec((1, D), lambda i: (0, 0)),
            pl.BlockSpec((D, D), lambda i: (0, 0)),
            pl.BlockSpec((1, D), lambda i: (0, 0)),
            pl.BlockSpec((1, D), lambda i: (0, 0)),
            pl.BlockSpec((1, D), lambda i: (0, 0)),
            pl.BlockSpec((D, D), lambda i: (0, 0)),
            pl.BlockSpec((D, A8), lambda i: (0, 0)),
        ],
        out_specs=[
            pl.BlockSpec((BN, D), lambda i: (i, 0)),
            pl.BlockSpec((BN, D), lambda i: (i, 0)),
            pl.BlockSpec((BN, A8), lambda i: (i, 0)),
            pl.BlockSpec((1, A8), lambda i: (0, 0)),
        ],
        out_shape=[
            jax.ShapeDtypeStruct((N, D), jnp.float32),
            jax.ShapeDtypeStruct((N, D), jnp.float32),
            jax.ShapeDtypeStruct((N, A8), jnp.float32),
            jax.ShapeDtypeStruct((1, A8), jnp.float32),
        ],
    )(eps.reshape(1, 1), x, aggr, gW1, gb1.reshape(1, D), gW2,
      gb2.reshape(1, D), n1g.reshape(1, D), n1b.reshape(1, D), gatW, AS)


def _gat_edges(xh, asd128, gmax16, src, dst, H):
    N, D = xh.shape
    E = src.shape[0]
    nchunks = E // _CH
    NQ = N // _NC
    rpt = (NQ // _NS) // 8 * 8
    rem = NQ - _NS * rpt
    ND4 = NQ * 4 + 16

    @functools.partial(
        pl.kernel,
        out_type=(
            jax.ShapeDtypeStruct((_NC, 1, NQ, D), jnp.float32),
            jax.ShapeDtypeStruct((_NC, 1, _NS, ND4), jnp.float32),
        ),
        mesh=_mesh(),
        scratch_types=[
            pltpu.VMEM((_CH,), jnp.int32),
            pltpu.VMEM((_CH,), jnp.int32),
            pltpu.VMEM((_CH,), jnp.int32),
            pltpu.VMEM((_CH, D), jnp.float32),
            pltpu.VMEM((_CH, 4), jnp.float32),
            pltpu.VMEM((_CH, D), jnp.float32),
            pltpu.VMEM((_CH, D), jnp.float32),
            pltpu.VMEM((16,), jnp.float32),
            pltpu.VMEM((ND4,), jnp.float32),
            pltpu.VMEM_SHARED((NQ + 8, D), jnp.float32),
            pltpu.SemaphoreType.DMA,
            pltpu.SemaphoreType.DMA,
            pltpu.SemaphoreType.DMA,
        ],
        compiler_params=pltpu.CompilerParams(needs_layout_passes=False),
    )
    def k(xh_hbm, asd_hbm, gm_hbm, src_hbm, dst_hbm,
          out_hbm, den_hbm,
          sidx, didx, didx2, xbuf, stg, arow_s, arow_d, gm_v, lden, acc,
          sem, sem2, sem3):
        c = lax.axis_index("c")
        s = lax.axis_index("s")
        off = pl.multiple_of(s * rpt, 8)
        pltpu.sync_copy(gm_hbm, gm_v)

        iota16 = lax.iota(jnp.int32, 16)
        zeros16 = jnp.zeros((16,), jnp.float32)
        ntr = (nchunks - s + _NS - 1) // _NS
        gmv = gm_v[...]

        def zxbuf(g, _):
            rows = iota16 + g * 16
            for cj in range(D):
                plsc.store_scatter(
                    xbuf, [rows, jnp.full((16,), cj, jnp.int32)], zeros16)
            return 0

        for q in range(1):
            lo = c * NQ

            lax.fori_loop(0, _CH // 16, zxbuf, 0)
            _zero_fill(acc, xbuf, off, rpt)

            @pl.when(s == _NS - 1)
            def _():
                _zero_fill(acc, xbuf, _NS * rpt, (NQ + 8) - _NS * rpt)

            def zden(g, _):
                lden[pl.ds(pl.multiple_of(g * 16, 16), 16)] = zeros16
                return 0

            lax.fori_loop(0, ND4 // 16, zden, 0)
            plsc.subcore_barrier()

            def chunk(kk, _):
                @pl.when(kk > 0)
                def _():
                    pltpu.make_async_copy(
                        xh_hbm.at[pl.ds(0, _CH)], xbuf, sem3).wait()

                base = pl.multiple_of((s + kk * _NS) * _CH, _CH)
                pltpu.sync_copy(src_hbm.at[pl.ds(base, _CH)], sidx)
                pltpu.sync_copy(dst_hbm.at[pl.ds(base, _CH)], didx)
                ga = pltpu.async_copy(asd_hbm.at[sidx], arow_s, sem2)
                gb = pltpu.async_copy(asd_hbm.at[didx], arow_d, sem2)
                gat = pltpu.async_copy(xh_hbm.at[sidx], xbuf, sem)
                for g in range(_CH // 16):
                    sl = pl.ds(g * 16, 16)
                    lv = didx[sl] - lo
                    oob = (lv < 0) | (lv >= NQ)
                    didx2[sl] = jnp.where(oob, NQ, lv)
                ga.wait()
                gb.wait()

                def exgrp(g, _2):
                    goff = pl.multiple_of(g * 16, 16)
                    rows = iota16 + g * 16
                    lv = didx2[pl.ds(goff, 16)]
                    for h in range(H):
                        va = plsc.load_gather(
                            arow_s, [rows, jnp.full((16,), h, jnp.int32)])
                        vb = plsc.load_gather(
                            arow_d, [rows, jnp.full((16,), H + h, jnp.int32)])
                        zl = _leaky(va + vb)
                        kl = _leaky(gmv[h] + vb)
                        ex = jnp.exp(zl - kl)
                        plsc.store_scatter(
                            stg, [rows, jnp.full((16,), h, jnp.int32)], ex)
                        plsc.addupdate_scatter(lden, [lv * 4 + h], ex)
                    return 0

                lax.fori_loop(0, _CH // 16, exgrp, 0)
                gat.wait()

                def scale(g, _2):
                    rows = iota16 + g * 16
                    for h in range(H):
                        att = plsc.load_gather(
                            stg, [rows, jnp.full((16,), h, jnp.int32)])
                        for cj in range(D // H):
                            cs = jnp.full((16,), h * (D // H) + cj, jnp.int32)
                            v = plsc.load_gather(xbuf, [rows, cs])
                            plsc.store_scatter(xbuf, [rows, cs], v * att)
                    return 0

                lax.fori_loop(0, _CH // 16, scale, 0)
                pltpu.async_copy(xbuf, acc.at[didx2], sem3, add=True)
                return 0

            lax.fori_loop(0, ntr, chunk, 0)
            pltpu.make_async_copy(xh_hbm.at[pl.ds(0, _CH)], xbuf, sem3).wait()
            plsc.subcore_barrier()
            pltpu.sync_copy(acc.at[pl.ds(off, rpt)],
                            out_hbm.at[c, q, pl.ds(off, rpt)])

            @pl.when(s == _NS - 1)
            def _():
                pltpu.sync_copy(acc.at[pl.ds(_NS * rpt, rem)],
                                out_hbm.at[c, q, pl.ds(_NS * rpt, rem)])

            pltpu.sync_copy(lden, den_hbm.at[c, q, s])
            plsc.subcore_barrier()

    return k(xh, asd128, gmax16, src, dst)


def _final(acc, dparts, asd, gmax, xh, h, gbias, n2g, n2b, R4, R16, H):
    N, D = xh.shape
    BN = 2000
    grid = N // BN
    A8 = asd.shape[1]
    P = dparts.shape[1]

    def body(a0, d0, asd_r, gm_r, xh_r, h_r, gb, g2, b2, r4, r16, o):
        asd_v = asd_r[...]
        a_s = lax.slice(asd_v, (0, 0), (BN, H))
        a_d = lax.slice(asd_v, (0, H), (BN, 2 * H))
        gm = lax.slice(gm_r[...], (0, 0), (1, H))
        ex = jnp.exp(_leaky(a_s + a_d) - _leaky(gm + a_d))
        den4 = jnp.dot(d0[...], r16[...],
                       preferred_element_type=jnp.float32) + ex
        r4v = r4[...]
        ex128 = jnp.dot(ex, r4v, preferred_element_type=jnp.float32)
        den128 = jnp.dot(den4, r4v, preferred_element_type=jnp.float32)
        out = a0[...] + ex128 * xh_r[...]
        g1 = out / (den128 + 1e-16) + gb[...]
        r = jnp.maximum(g1, 0.0)
        m = jnp.mean(r, axis=-1, keepdims=True)
        v = jnp.mean((r - m) * (r - m), axis=-1, keepdims=True)
        o[...] = (r - m) / jnp.sqrt(v + 1e-5) * g2[...] + b2[...] + h_r[...]

    return pl.pallas_call(
        body,
        grid=(grid,),
        in_specs=[
            pl.BlockSpec((BN, D), lambda i: (i, 0)),
            pl.BlockSpec((BN, P), lambda i: (i, 0)),
            pl.BlockSpec((BN, A8), lambda i: (i, 0)),
            pl.BlockSpec((1, A8), lambda i: (0, 0)),
            pl.BlockSpec((BN, D), lambda i: (i, 0)),
            pl.BlockSpec((BN, D), lambda i: (i, 0)),
            pl.BlockSpec((1, D), lambda i: (0, 0)),
            pl.BlockSpec((1, D), lambda i: (0, 0)),
            pl.BlockSpec((1, D), lambda i: (0, 0)),
            pl.BlockSpec((H, D), lambda i: (0, 0)),
            pl.BlockSpec((P, H), lambda i: (0, 0)),
        ],
        out_specs=pl.BlockSpec((BN, D), lambda i: (i, 0)),
        out_shape=jax.ShapeDtypeStruct((N, D), jnp.float32),
    )(acc, dparts, asd, gmax, xh, h, gbias.reshape(1, D),
      n2g.reshape(1, D), n2b.reshape(1, D), R4, R16)


def kernel(x, edge_index, edge_attr, eps, geW, geb, gW1, gb1, gW2, gb2,
           n1g, n1b, gatW, asrc, adst, gbias, n2g, n2b):
    N, D = x.shape
    H, C = asrc.shape
    NH = N // _NC
    src = edge_index[0]
    dst = edge_index[1]

    rows = jnp.arange(D)
    hh = rows // C
    cc = rows % C
    AS = jnp.zeros((D, D), jnp.float32)
    AS = AS.at[rows, hh].set(asrc[hh, cc])
    AS = AS.at[rows, H + hh].set(adst[hh, cc])
    R4 = (jnp.arange(H)[:, None] == hh[None, :]).astype(jnp.float32)
    R16 = jnp.tile(jnp.eye(H, dtype=jnp.float32), (_NS, 1))

    e = _edge_linear(edge_attr, geW, geb)
    aggr2 = _gine_aggregate(x, e, src, dst)
    aggr = aggr2.reshape(N, D)
    h, xh, asd, gmax = _node_stage(x, aggr, eps, gW1, gb1, gW2,
                                   gb2, n1g, n1b, gatW, AS)
    gmax16 = gmax[0, :16]
    outp, denp = _gat_edges(xh, asd, gmax16, src, dst, H)
    outacc = outp.reshape(N, D)
    nc_, qp_, ns_, nd4_ = denp.shape
    np_ = (nd4_ - 16) // 4
    dparts = jnp.transpose(
        denp[:, :, :, :np_ * 4].reshape(nc_, qp_, ns_, np_, H),
        (0, 1, 3, 2, 4)
    ).reshape(N, ns_ * H)
    return _final(outacc, dparts, asd, gmax, xh, h, gbias, n2g, n2b,
                  R4, R16, H)


# speedup vs baseline: 6.9173x; 1.0146x over previous
import functools

import jax
import jax.numpy as jnp
from jax import lax
from jax.experimental import pallas as pl
from jax.experimental.pallas import tpu as pltpu
from jax.experimental.pallas import tpu_sc as plsc

_NC = 2
_NS = 16
_CHA = 64
_CH = 80


def _leaky(z):
    return jnp.where(z >= 0.0, z, 0.2 * z)


def _mesh():
    return plsc.VectorSubcoreMesh(
        core_axis_name="c", subcore_axis_name="s", num_cores=_NC,
        num_subcores=_NS)


def _edge_linear(edge_attr, geW, geb):
    E, D = edge_attr.shape
    BE = 3200
    grid = E // BE

    def body(ea, w, b, o):
        o[...] = (
            jnp.dot(ea[...], w[...], preferred_element_type=jnp.float32) + b[...]
        )

    return pl.pallas_call(
        body,
        grid=(grid,),
        in_specs=[
            pl.BlockSpec((BE, D), lambda i: (i, 0)),
            pl.BlockSpec((D, D), lambda i: (0, 0)),
            pl.BlockSpec((1, D), lambda i: (0, 0)),
        ],
        out_specs=pl.BlockSpec((BE, D), lambda i: (i, 0)),
        out_shape=jax.ShapeDtypeStruct((E, D), jnp.float32),
    )(edge_attr, geW, geb.reshape(1, D))


def _zero_fill(dst_ref, zbuf, off, total):
    done = 0
    zrows = zbuf.shape[0]
    while done < total:
        step = min(zrows, total - done)
        pltpu.sync_copy(zbuf.at[pl.ds(0, step)],
                        dst_ref.at[pl.ds(pl.multiple_of(off + done, 8), step)])
        done += step


def _gine_aggregate(x, e, src, dst):
    N, D = x.shape
    E = e.shape[0]
    nchunks = E // _CHA
    NQ = N // (2 * _NC)
    rpt = (NQ // _NS) // 8 * 8
    rem = NQ - _NS * rpt

    @functools.partial(
        pl.kernel,
        out_type=jax.ShapeDtypeStruct((_NC, 2, NQ, D), jnp.float32),
        mesh=_mesh(),
        scratch_types=[
            pltpu.VMEM((_CHA,), jnp.int32),
            pltpu.VMEM((_CHA,), jnp.int32),
            pltpu.VMEM((_CHA, D), jnp.float32),
            pltpu.VMEM((_CHA, D), jnp.float32),
            pltpu.VMEM_SHARED((NQ + 8, D), jnp.float32),
            pltpu.SemaphoreType.DMA,
            pltpu.SemaphoreType.DMA,
        ],
    )
    def k(x_hbm, e_hbm, src_hbm, dst_hbm, out_hbm,
          sidx, didx, ebuf, xbuf, acc, sem, sem3):
        c = lax.axis_index("c")
        s = lax.axis_index("s")
        off = pl.multiple_of(s * rpt, 8)
        ntr = (nchunks - s + _NS - 1) // _NS

        for q in range(2):
            lo = c * (2 * NQ) + q * NQ

            def zrow(i, _):
                for j in range(D // 16):
                    ebuf[i, pl.ds(j * 16, 16)] = jnp.zeros((16,), jnp.float32)
                return 0

            lax.fori_loop(0, _CHA, zrow, 0)
            _zero_fill(acc, ebuf, off, rpt)

            @pl.when(s == _NS - 1)
            def _():
                _zero_fill(acc, ebuf, _NS * rpt, (NQ + 8) - _NS * rpt)

            plsc.subcore_barrier()

            def chunk(kk, _):
                @pl.when(kk > 0)
                def _():
                    pltpu.make_async_copy(
                        e_hbm.at[pl.ds(0, _CHA)], ebuf, sem3).wait()

                base = pl.multiple_of((s + kk * _NS) * _CHA, _CHA)
                pltpu.sync_copy(src_hbm.at[pl.ds(base, _CHA)], sidx)
                pltpu.sync_copy(dst_hbm.at[pl.ds(base, _CHA)], didx)
                gat = pltpu.async_copy(x_hbm.at[sidx], xbuf, sem)
                pltpu.sync_copy(e_hbm.at[pl.ds(base, _CHA)], ebuf)
                for g in range(_CHA // 16):
                    sl = pl.ds(g * 16, 16)
                    lv = didx[sl] - lo
                    oob = (lv < 0) | (lv >= NQ)
                    didx[sl] = jnp.where(oob, NQ, lv)
                gat.wait()

                def row(i, _2):
                    for j in range(D // 16):
                        sl = pl.ds(j * 16, 16)
                        ebuf[i, sl] = jnp.maximum(
                            ebuf[i, sl] + xbuf[i, sl], 0.0)
                    return 0

                lax.fori_loop(0, _CHA, row, 0)
                pltpu.async_copy(ebuf, acc.at[didx], sem3, add=True)
                return 0

            lax.fori_loop(0, ntr, chunk, 0)
            pltpu.make_async_copy(e_hbm.at[pl.ds(0, _CHA)], ebuf, sem3).wait()
            plsc.subcore_barrier()
            pltpu.sync_copy(acc.at[pl.ds(off, rpt)],
                            out_hbm.at[c, q, pl.ds(off, rpt)])

            @pl.when(s == _NS - 1)
            def _():
                pltpu.sync_copy(acc.at[pl.ds(_NS * rpt, rem)],
                                out_hbm.at[c, q, pl.ds(_NS * rpt, rem)])

            plsc.subcore_barrier()

    return k(x, e, src, dst)


def _node_stage(x, aggr, eps, gW1, gb1, gW2, gb2, n1g, n1b, gatW, AS):
    N, D = x.shape
    BN = 2000
    grid = N // BN
    A8 = AS.shape[1]

    def body(eps_r, x_r, ag_r, w1, b1, w2, b2, g1, bb1, gw, as_r,
             h_o, xh_o, asd_o, gmax_o):
        i = pl.program_id(0)
        xv = x_r[...]
        hpre = (1.0 + eps_r[0, 0]) * xv + ag_r[...]
        t = jnp.maximum(
            jnp.dot(hpre, w1[...], preferred_element_type=jnp.float32)
            + b1[...], 0.0)
        h2 = jnp.dot(t, w2[...], preferred_element_type=jnp.float32) + b2[...]
        r = jnp.maximum(h2, 0.0)
        m = jnp.mean(r, axis=-1, keepdims=True)
        v = jnp.mean((r - m) * (r - m), axis=-1, keepdims=True)
        hn = (r - m) / jnp.sqrt(v + 1e-5) * g1[...] + bb1[...]
        h = hn + xv
        h_o[...] = h
        xh = jnp.dot(h, gw[...], preferred_element_type=jnp.float32)
        xh_o[...] = xh
        asd = jnp.dot(xh, as_r[...], preferred_element_type=jnp.float32)
        asd_o[...] = asd
        blkmax = jnp.max(asd, axis=0, keepdims=True)

        @pl.when(i == 0)
        def _():
            gmax_o[...] = jnp.full((1, A8), -1e30, jnp.float32)

        gmax_o[...] = jnp.maximum(gmax_o[...], blkmax)

    return pl.pallas_call(
        body,
        grid=(grid,),
        in_specs=[
            pl.BlockSpec(memory_space=pltpu.SMEM),
            pl.BlockSpec((BN, D), lambda i: (i, 0)),
            pl.BlockSpec((BN, D), lambda i: (i, 0)),
            pl.BlockSpec((D, D), lambda i: (0, 0)),
            pl.BlockSpec((1, D), lambda i: (0, 0)),
            pl.BlockSpec((D, D), lambda i: (0, 0)),
            pl.BlockSpec((1, D), lambda i: (0, 0)),
            pl.BlockSpec((1, D), lambda i: (0, 0)),
            pl.BlockSpec((1, D), lambda i: (0, 0)),
            pl.BlockSpec((D, D), lambda i: (0, 0)),
            pl.BlockSpec((D, A8), lambda i: (0, 0)),
        ],
        out_specs=[
            pl.BlockSpec((BN, D), lambda i: (i, 0)),
            pl.BlockSpec((BN, D), lambda i: (i, 0)),
            pl.BlockSpec((BN, A8), lambda i: (i, 0)),
            pl.BlockSpec((1, A8), lambda i: (0, 0)),
        ],
        out_shape=[
            jax.ShapeDtypeStruct((N, D), jnp.float32),
            jax.ShapeDtypeStruct((N, D), jnp.float32),
            jax.ShapeDtypeStruct((N, A8), jnp.float32),
            jax.ShapeDtypeStruct((1, A8), jnp.float32),
        ],
    )(eps.reshape(1, 1), x, aggr, gW1, gb1.reshape(1, D), gW2,
      gb2.reshape(1, D), n1g.reshape(1, D), n1b.reshape(1, D), gatW, AS)


def _gat_edges(xh, asd128, gmax16, src, dst, H):
    N, D = xh.shape
    E = src.shape[0]
    nchunks = E // _CH
    NQ = N // _NC
    rpt = (NQ // _NS) // 8 * 8
    rem = NQ - _NS * rpt
    ND4 = NQ * 4 + 16

    @functools.partial(
        pl.kernel,
        out_type=(
            jax.ShapeDtypeStruct((_NC, 1, NQ, D), jnp.float32),
            jax.ShapeDtypeStruct((_NC, 1, _NS, ND4), jnp.float32),
        ),
        mesh=_mesh(),
        scratch_types=[
            pltpu.VMEM((_CH,), jnp.int32),
            pltpu.VMEM((_CH,), jnp.int32),
            pltpu.VMEM((_CH,), jnp.int32),
            pltpu.VMEM((_CH, D), jnp.float32),
            pltpu.VMEM((_CH, 4), jnp.float32),
            pltpu.VMEM((_CH, D), jnp.float32),
            pltpu.VMEM((_CH, D), jnp.float32),
            pltpu.VMEM((16,), jnp.float32),
            pltpu.VMEM((ND4,), jnp.float32),
            pltpu.VMEM_SHARED((NQ + 8, D), jnp.float32),
            pltpu.SemaphoreType.DMA,
            pltpu.SemaphoreType.DMA,
            pltpu.SemaphoreType.DMA,
        ],
        compiler_params=pltpu.CompilerParams(needs_layout_passes=False),
    )
    def k(xh_hbm, asd_hbm, gm_hbm, src_hbm, dst_hbm,
          out_hbm, den_hbm,
          sidx, didx, didx2, xbuf, stg, arow_s, arow_d, gm_v, lden, acc,
          sem, sem2, sem3):
        c = lax.axis_index("c")
        s = lax.axis_index("s")
        off = pl.multiple_of(s * rpt, 8)
        pltpu.sync_copy(gm_hbm, gm_v)

        iota16 = lax.iota(jnp.int32, 16)
        zeros16 = jnp.zeros((16,), jnp.float32)
        ntr = (nchunks - s + _NS - 1) // _NS
        gmv = gm_v[...]

        def zxbuf(g, _):
            rows = iota16 + g * 16
            for cj in range(D):
                plsc.store_scatter(
                    xbuf, [rows, jnp.full((16,), cj, jnp.int32)], zeros16)
            return 0

        for q in range(1):
            lo = c * NQ

            lax.fori_loop(0, _CH // 16, zxbuf, 0)
            _zero_fill(acc, xbuf, off, rpt)

            @pl.when(s == _NS - 1)
            def _():
                _zero_fill(acc, xbuf, _NS * rpt, (NQ + 8) - _NS * rpt)

            def zden(g, _):
                lden[pl.ds(pl.multiple_of(g * 16, 16), 16)] = zeros16
                return 0

            lax.fori_loop(0, ND4 // 16, zden, 0)
            plsc.subcore_barrier()

            def chunk(kk, _):
                @pl.when(kk > 0)
                def _():
                    pltpu.make_async_copy(
                        xh_hbm.at[pl.ds(0, _CH)], xbuf, sem3).wait()

                base = pl.multiple_of((s + kk * _NS) * _CH, _CH)
                pltpu.sync_copy(src_hbm.at[pl.ds(base, _CH)], sidx)
                pltpu.sync_copy(dst_hbm.at[pl.ds(base, _CH)], didx)
                ga = pltpu.async_copy(asd_hbm.at[sidx], arow_s, sem2)
                gb = pltpu.async_copy(asd_hbm.at[didx], arow_d, sem2)
                gat = pltpu.async_copy(xh_hbm.at[sidx], xbuf, sem)
                for g in range(_CH // 16):
                    sl = pl.ds(g * 16, 16)
                    lv = didx[sl] - lo
                    oob = (lv < 0) | (lv >= NQ)
                    didx2[sl] = jnp.where(oob, NQ, lv)
                ga.wait()
                gb.wait()

                def exgrp(g, _2):
                    goff = pl.multiple_of(g * 16, 16)
                    rows = iota16 + g * 16
                    lv = didx2[pl.ds(goff, 16)]
                    for h in range(H):
                        va = plsc.load_gather(
                            arow_s, [rows, jnp.full((16,), h, jnp.int32)])
                        vb = plsc.load_gather(
                            arow_d, [rows, jnp.full((16,), H + h, jnp.int32)])
                        zl = _leaky(va + vb)
                        kl = _leaky(gmv[h] + vb)
                        ex = jnp.exp(zl - kl)
                        plsc.store_scatter(
                            stg, [rows, jnp.full((16,), h, jnp.int32)], ex)
                        plsc.addupdate_scatter(lden, [lv * 4 + h], ex)
                    return 0

                lax.fori_loop(0, _CH // 16, exgrp, 0)
                gat.wait()

                def scale(g, _2):
                    rows = iota16 + g * 16
                    for h in range(H):
                        att = plsc.load_gather(
                            stg, [rows, jnp.full((16,), h, jnp.int32)])
                        for cj in range(D // H):
                            cs = jnp.full((16,), h * (D // H) + cj, jnp.int32)
                            v = plsc.load_gather(xbuf, [rows, cs])
                            plsc.store_scatter(xbuf, [rows, cs], v * att)
                    return 0

                lax.fori_loop(0, _CH // 16, scale, 0)
                pltpu.async_copy(xbuf, acc.at[didx2], sem3, add=True)
                return 0

            lax.fori_loop(0, ntr, chunk, 0)
            pltpu.make_async_copy(xh_hbm.at[pl.ds(0, _CH)], xbuf, sem3).wait()
            plsc.subcore_barrier()
            pltpu.sync_copy(acc.at[pl.ds(off, rpt)],
                            out_hbm.at[c, q, pl.ds(off, rpt)])

            @pl.when(s == _NS - 1)
            def _():
                pltpu.sync_copy(acc.at[pl.ds(_NS * rpt, rem)],
                                out_hbm.at[c, q, pl.ds(_NS * rpt, rem)])

            pltpu.sync_copy(lden, den_hbm.at[c, q, s])
            plsc.subcore_barrier()

    return k(xh, asd128, gmax16, src, dst)


def _final(acc, dparts, asd, gmax, xh, h, gbias, n2g, n2b, R4, R16, H):
    N, D = xh.shape
    BN = 2000
    grid = N // BN
    A8 = asd.shape[1]
    P = dparts.shape[1]

    def body(a0, d0, asd_r, gm_r, xh_r, h_r, gb, g2, b2, r4, r16, o):
        asd_v = asd_r[...]
        a_s = lax.slice(asd_v, (0, 0), (BN, H))
        a_d = lax.slice(asd_v, (0, H), (BN, 2 * H))
        gm = lax.slice(gm_r[...], (0, 0), (1, H))
        ex = jnp.exp(_leaky(a_s + a_d) - _leaky(gm + a_d))
        den4 = jnp.dot(d0[...], r16[...],
                       preferred_element_type=jnp.float32) + ex
        r4v = r4[...]
        ex128 = jnp.dot(ex, r4v, preferred_element_type=jnp.float32)
        den128 = jnp.dot(den4, r4v, preferred_element_type=jnp.float32)
        out = a0[...] + ex128 * xh_r[...]
        g1 = out / (den128 + 1e-16) + gb[...]
        r = jnp.maximum(g1, 0.0)
        m = jnp.mean(r, axis=-1, keepdims=True)
        v = jnp.mean((r - m) * (r - m), axis=-1, keepdims=True)
        o[...] = (r - m) / jnp.sqrt(v + 1e-5) * g2[...] + b2[...] + h_r[...]

    return pl.pallas_call(
        body,
        grid=(grid,),
        in_specs=[
            pl.BlockSpec((BN, D), lambda i: (i, 0)),
            pl.BlockSpec((BN, P), lambda i: (i, 0)),
            pl.BlockSpec((BN, A8), lambda i: (i, 0)),
            pl.BlockSpec((1, A8), lambda i: (0, 0)),
            pl.BlockSpec((BN, D), lambda i: (i, 0)),
            pl.BlockSpec((BN, D), lambda i: (i, 0)),
            pl.BlockSpec((1, D), lambda i: (0, 0)),
            pl.BlockSpec((1, D), lambda i: (0, 0)),
            pl.BlockSpec((1, D), lambda i: (0, 0)),
            pl.BlockSpec((H, D), lambda i: (0, 0)),
            pl.BlockSpec((P, H), lambda i: (0, 0)),
        ],
        out_specs=pl.BlockSpec((BN, D), lambda i: (i, 0)),
        out_shape=jax.ShapeDtypeStruct((N, D), jnp.float32),
    )(acc, dparts, asd, gmax, xh, h, gbias.reshape(1, D),
      n2g.reshape(1, D), n2b.reshape(1, D), R4, R16)


def kernel(x, edge_index, edge_attr, eps, geW, geb, gW1, gb1, gW2, gb2,
           n1g, n1b, gatW, asrc, adst, gbias, n2g, n2b):
    N, D = x.shape
    H, C = asrc.shape
    NH = N // _NC
    src = edge_index[0]
    dst = edge_index[1]

    rows = jnp.arange(D)
    hh = rows // C
    cc = rows % C
    AS = jnp.zeros((D, D), jnp.float32)
    AS = AS.at[rows, hh].set(asrc[hh, cc])
    AS = AS.at[rows, H + hh].set(adst[hh, cc])
    R4 = (jnp.arange(H)[:, None] == hh[None, :]).astype(jnp.float32)
    R16 = jnp.tile(jnp.eye(H, dtype=jnp.float32), (_NS, 1))

    e = _edge_linear(edge_attr, geW, geb)
    aggr2 = _gine_aggregate(x, e, src, dst)
    aggr = aggr2.reshape(N, D)
    h, xh, asd, gmax = _node_stage(x, aggr, eps, gW1, gb1, gW2,
                                   gb2, n1g, n1b, gatW, AS)
    gmax16 = gmax[0, :16]
    outp, denp = _gat_edges(xh, asd, gmax16, src, dst, H)
    outacc = outp.reshape(N, D)
    nc_, qp_, ns_, nd4_ = denp.shape
    np_ = (nd4_ - 16) // 4
    dparts = jnp.transpose(
        denp[:, :, :, :np_ * 4].reshape(nc_, qp_, ns_, np_, H),
        (0, 1, 3, 2, 4)
    ).reshape(N, ns_ * H)
    return _final(outacc, dparts, asd, gmax, xh, h, gbias, n2g, n2b,
                  R4, R16, H)
